# Initial kernel scaffold; baseline (speedup 1.0000x reference)
import jax
import jax.numpy as jnp
from jax.experimental import pallas as pl


def kernel(x, edge_index, W_src, W_dst, double_attn, bias, prelu_a):
    raise NotImplementedError("write your pallas kernel here")


# speedup vs baseline: 24.2932x; 24.2932x over previous
import dataclasses
import functools

import jax
import jax.numpy as jnp
import numpy as np
from jax import lax
from jax.experimental import pallas as pl
from jax.experimental.pallas import tpu as pltpu
from jax.experimental.pallas import tpu_sc as plsc

N = 10000
E = 320000
F = 128
H = 4
D = 32

NC = 2
NS = 16
NW = NC * NS
EPW = E // NW
G = 16
NGROUPS = EPW // G
ZCH = 40
NCHUNK = N // ZCH

_LEAKY = 0.2
_EPS = 1e-16


def _proj_body(x_ref, ws_ref, wd_ref, sp_ref, dp_ref):
    xb = x_ref[...]
    sp_ref[...] = lax.dot_general(
        xb, ws_ref[...], (((1,), (1,)), ((), ())),
        preferred_element_type=jnp.float32)
    dp_ref[...] = lax.dot_general(
        xb, wd_ref[...], (((1,), (1,)), ((), ())),
        preferred_element_type=jnp.float32)


def _final_body(num_ref, den_ref, x_ref, bias_ref, m_ref, pa_ref, o_ref):
    num = num_ref[0] + num_ref[1]
    den = den_ref[0] + den_ref[1]
    denb = lax.dot_general(
        den, m_ref[...], (((1,), (0,)), ((), ())),
        preferred_element_type=jnp.float32)
    o = num / (denb + _EPS) + x_ref[...] + bias_ref[...]
    pa = pa_ref[...]
    o_ref[...] = jnp.where(o >= 0, o, pa * o)


def _gat_edge_kernel(sp_hbm, dp_hbm, sidx_hbm, didx_hbm, a_hbm,
                     num_out, den_out,
                     sidx_buf, didx_buf, srows, drows, msg, att2, pbuf,
                     abuf, zbuf, dzbuf, num_acc, den_acc):
    cid = lax.axis_index("c")
    sid = lax.axis_index("s")
    wid = cid * NS + sid

    fzero = jnp.zeros((16,), jnp.float32)
    iota = lax.iota(jnp.int32, 16)
    ibase = iota * 16

    @pl.loop(0, ZCH)
    def _(r):
        for v in range(8):
            zbuf[r, pl.ds(v * 16, 16)] = fzero

    @pl.loop(0, ZCH)
    def _(r):
        dzbuf[r, :] = fzero

    @pl.loop(0, NCHUNK)
    def _(c):
        @pl.when(c % NS == sid)
        def _():
            pltpu.sync_copy(zbuf, num_acc.at[pl.ds(c * ZCH, ZCH)])
            pltpu.sync_copy(dzbuf, den_acc.at[pl.ds(c * ZCH, ZCH)])

    for j in range(G):
        att2[j, :] = fzero

    pltpu.sync_copy(sidx_hbm.at[wid], sidx_buf)
    pltpu.sync_copy(didx_hbm.at[wid], didx_buf)
    pltpu.sync_copy(a_hbm, abuf)
    av = [abuf[pl.ds(v * 16, 16)] for v in range(8)]
    cvec = [jnp.full((16,), j, jnp.int32) for j in range(G)]
    hvec = [jnp.full((16,), h, jnp.int32) for h in range(H)]

    gdn = jax.lax.GatherDimensionNumbers(
        offset_dims=(), collapsed_slice_dims=(0,), start_index_map=(0,))

    plsc.subcore_barrier()

    @pl.loop(0, NGROUPS)
    def _(g):
        sidx_vec = sidx_buf[g]
        didx_vec = didx_buf[g]
        pltpu.sync_copy(sp_hbm.at[sidx_vec], srows)
        pltpu.sync_copy(dp_hbm.at[didx_vec], drows)

        for j in range(G):
            for h in range(H):
                ph = None
                for vv in range(2):
                    v = 2 * h + vv
                    z = srows[j, pl.ds(v * 16, 16)] + drows[j, pl.ds(v * 16, 16)]
                    lk = jnp.maximum(z, _LEAKY * z)
                    t = lk * av[v]
                    ph = t if ph is None else ph + t
                plsc.store_scatter(pbuf, [ibase + (h * 256 + j)], ph)

        attns = []
        for h in range(H):
            acc = fzero
            for l in range(16):
                acc = acc + pbuf[pl.ds(h * 256 + l * 16, 16)]
            attn = jnp.exp(acc)
            attns.append(attn)
            plsc.store_scatter(att2, [iota, hvec[h]], attn)

        for j in range(G):
            b = [lax.gather(attns[h], cvec[j][:, None], gdn, (1,),
                            mode=lax.GatherScatterMode.PROMISE_IN_BOUNDS)
                 for h in range(H)]
            for v in range(8):
                msg[j, pl.ds(v * 16, 16)] = (
                    srows[j, pl.ds(v * 16, 16)] * b[v // 2])

        pltpu.sync_copy(msg, num_acc.at[didx_vec], add=True)
        pltpu.sync_copy(att2, den_acc.at[didx_vec], add=True)

    plsc.subcore_barrier()

    @pl.loop(0, NCHUNK)
    def _(c):
        @pl.when(c % NS == sid)
        def _():
            pltpu.sync_copy(num_acc.at[pl.ds(c * ZCH, ZCH)], zbuf)
            pltpu.sync_copy(zbuf, num_out.at[cid, pl.ds(c * ZCH, ZCH)])
            pltpu.sync_copy(den_acc.at[pl.ds(c * ZCH, ZCH)], dzbuf)
            pltpu.sync_copy(dzbuf, den_out.at[cid, pl.ds(c * ZCH, ZCH)])


_HEAD_BCAST = np.zeros((16, 128), np.float32)
for _h in range(H):
    _HEAD_BCAST[_h, _h * D:(_h + 1) * D] = 1.0


@jax.jit
def kernel(x, edge_index, W_src, W_dst, double_attn, bias, prelu_a):
    src2d = edge_index[0].reshape(NW, NGROUPS, 16)
    dst2d = edge_index[1].reshape(NW, NGROUPS, 16)
    a_flat = double_attn.reshape(H * D)

    PB = 400
    sp, dp = pl.pallas_call(
        _proj_body,
        grid=(N // PB,),
        in_specs=[
            pl.BlockSpec((PB, F), lambda i: (i, 0)),
            pl.BlockSpec((H * D, F), lambda i: (0, 0)),
            pl.BlockSpec((H * D, F), lambda i: (0, 0)),
        ],
        out_specs=[
            pl.BlockSpec((PB, H * D), lambda i: (i, 0)),
            pl.BlockSpec((PB, H * D), lambda i: (i, 0)),
        ],
        out_shape=[
            jax.ShapeDtypeStruct((N, H * D), jnp.float32),
            jax.ShapeDtypeStruct((N, H * D), jnp.float32),
        ],
    )(x, W_src, W_dst)

    mesh = plsc.VectorSubcoreMesh(core_axis_name="c", subcore_axis_name="s")
    cp = pltpu.CompilerParams()
    if "needs_layout_passes" in pltpu.CompilerParams.__dataclass_fields__:
        cp = dataclasses.replace(cp, needs_layout_passes=False)
    if "use_tc_tiling_on_sc" in pltpu.CompilerParams.__dataclass_fields__:
        cp = dataclasses.replace(cp, use_tc_tiling_on_sc=False)
    sc_kernel = functools.partial(
        pl.kernel,
        compiler_params=cp,
        out_type=[
            jax.ShapeDtypeStruct((NC, N, H * D), jnp.float32),
            jax.ShapeDtypeStruct((NC, N, 16), jnp.float32),
        ],
        mesh=mesh,
        scratch_types=[
            pltpu.VMEM((NGROUPS, 16), jnp.int32),
            pltpu.VMEM((NGROUPS, 16), jnp.int32),
            pltpu.VMEM((G, H * D), jnp.float32),
            pltpu.VMEM((G, H * D), jnp.float32),
            pltpu.VMEM((G, H * D), jnp.float32),
            pltpu.VMEM((G, 16), jnp.float32),
            pltpu.VMEM((H * 256,), jnp.float32),
            pltpu.VMEM((H * D,), jnp.float32),
            pltpu.VMEM((ZCH, H * D), jnp.float32),
            pltpu.VMEM((ZCH, 16), jnp.float32),
            pltpu.VMEM_SHARED((N, H * D), jnp.float32),
            pltpu.VMEM_SHARED((N, 16), jnp.float32),
        ],
    )(_gat_edge_kernel)
    num_part, den_part = sc_kernel(sp, dp, src2d, dst2d, a_flat)

    bias_row = bias.reshape(1, H * D)
    pa_row = jnp.broadcast_to(prelu_a.reshape(1, 1), (1, H * D))
    m = jnp.asarray(_HEAD_BCAST)
    out = pl.pallas_call(
        _final_body,
        grid=(N // PB,),
        in_specs=[
            pl.BlockSpec((NC, PB, H * D), lambda i: (0, i, 0)),
            pl.BlockSpec((NC, PB, 16), lambda i: (0, i, 0)),
            pl.BlockSpec((PB, F), lambda i: (i, 0)),
            pl.BlockSpec((1, H * D), lambda i: (0, 0)),
            pl.BlockSpec((16, H * D), lambda i: (0, 0)),
            pl.BlockSpec((1, H * D), lambda i: (0, 0)),
        ],
        out_specs=pl.BlockSpec((PB, H * D), lambda i: (i, 0)),
        out_shape=jax.ShapeDtypeStruct((N, H * D), jnp.float32),
    )(num_part, den_part, x, bias_row, m, pa_row)
    return out


# speedup vs baseline: 35.5997x; 1.4654x over previous
import dataclasses
import functools

import jax
import jax.numpy as jnp
import numpy as np
from jax import lax
from jax.experimental import pallas as pl
from jax.experimental.pallas import tpu as pltpu
from jax.experimental.pallas import tpu_sc as plsc

N = 10000
E = 320000
F = 128
H = 4
D = 32

NC = 2
NS = 16
NW = NC * NS
EPW = E // NW
G = 16
NGROUPS = EPW // G
NGP = NGROUPS + 1
ZCH = 40
NP = N + ZCH
NCHUNK = NP // ZCH
NB = 2

_LEAKY = 0.2
_EPS = 1e-16


def _proj_body(x_ref, ws_ref, wd_ref, sp_ref, dp_ref):
    xb = x_ref[...]
    sp_ref[...] = lax.dot_general(
        xb, ws_ref[...], (((1,), (1,)), ((), ())),
        preferred_element_type=jnp.float32)
    dp_ref[...] = lax.dot_general(
        xb, wd_ref[...], (((1,), (1,)), ((), ())),
        preferred_element_type=jnp.float32)


def _final_body(num_ref, den_ref, x_ref, bias_ref, m_ref, pa_ref, o_ref):
    num = num_ref[0] + num_ref[1]
    den = den_ref[0] + den_ref[1]
    denb = lax.dot_general(
        den, m_ref[...], (((1,), (0,)), ((), ())),
        preferred_element_type=jnp.float32)
    o = num / (denb + _EPS) + x_ref[...] + bias_ref[...]
    pa = pa_ref[...]
    o_ref[...] = jnp.where(o >= 0, o, pa * o)


def _gat_edge_kernel(sp_hbm, dp_hbm, sidx_hbm, didx_hbm, a_hbm,
                     num_out, den_out,
                     sidx_buf, didx_buf, srows0, srows1, drows0, drows1,
                     msg0, msg1, att0, att1, pbuf,
                     abuf, zbuf, dzbuf, num_acc, den_acc,
                     gs0, gs1, gd0, gd1, sm0, sm1, sa0, sa1):
    cid = lax.axis_index("c")
    sid = lax.axis_index("s")
    wid = cid * NS + sid

    srows = [srows0, srows1]
    drows = [drows0, drows1]
    msg = [msg0, msg1]
    att2 = [att0, att1]
    gs = [gs0, gs1]
    gd = [gd0, gd1]
    sm = [sm0, sm1]
    sa = [sa0, sa1]

    fzero = jnp.zeros((16,), jnp.float32)
    iota = lax.iota(jnp.int32, 16)
    ibase = iota * 16

    @pl.loop(0, ZCH)
    def _(r):
        for v in range(8):
            zbuf[r, pl.ds(v * 16, 16)] = fzero

    @pl.loop(0, ZCH)
    def _(r):
        dzbuf[r, :] = fzero

    @pl.loop(0, NCHUNK)
    def _(c):
        @pl.when(c % NS == sid)
        def _():
            pltpu.sync_copy(zbuf, num_acc.at[pl.ds(c * ZCH, ZCH)])
            pltpu.sync_copy(dzbuf, den_acc.at[pl.ds(c * ZCH, ZCH)])

    for b in range(NB):
        for j in range(G):
            att2[b][j, :] = fzero

    pltpu.sync_copy(sidx_hbm.at[wid], sidx_buf)
    pltpu.sync_copy(didx_hbm.at[wid], didx_buf)
    pltpu.sync_copy(a_hbm, abuf)
    av = [abuf[pl.ds(v * 16, 16)] for v in range(8)]
    cvec = [jnp.full((16,), j, jnp.int32) for j in range(G)]
    hvec = [jnp.full((16,), h, jnp.int32) for h in range(H)]

    gdn = jax.lax.GatherDimensionNumbers(
        offset_dims=(), collapsed_slice_dims=(0,), start_index_map=(0,))

    plsc.subcore_barrier()

    def compute_group(b):
        for j in range(G):
            for h in range(H):
                ph = None
                for vv in range(2):
                    v = 2 * h + vv
                    z = (srows[b][j, pl.ds(v * 16, 16)]
                         + drows[b][j, pl.ds(v * 16, 16)])
                    lk = jnp.maximum(z, _LEAKY * z)
                    t = lk * av[v]
                    ph = t if ph is None else ph + t
                plsc.store_scatter(pbuf, [ibase + (h * 256 + j)], ph)

        attns = []
        for h in range(H):
            acc = fzero
            for l in range(16):
                acc = acc + pbuf[pl.ds(h * 256 + l * 16, 16)]
            attn = jnp.exp(acc)
            attns.append(attn)
            plsc.store_scatter(att2[b], [iota, hvec[h]], attn)

        for j in range(G):
            bc = [lax.gather(attns[h], cvec[j][:, None], gdn, (1,),
                             mode=lax.GatherScatterMode.PROMISE_IN_BOUNDS)
                  for h in range(H)]
            for v in range(8):
                msg[b][j, pl.ds(v * 16, 16)] = (
                    srows[b][j, pl.ds(v * 16, 16)] * bc[v // 2])

    for b in range(NB):
        pltpu.async_copy(sp_hbm.at[sidx_buf[b]], srows[b], gs[b])
        pltpu.async_copy(dp_hbm.at[didx_buf[b]], drows[b], gd[b])

    @pl.loop(0, NGP, step=NB)
    def _(g):
        for b in range(NB):
            gb = g + b
            sidx_vec = sidx_buf[gb]
            didx_vec = didx_buf[gb]

            @pl.when(gb >= NB)
            def _():
                od = didx_buf[gb - NB]
                pltpu.make_async_copy(msg[b], num_acc.at[od], sm[b]).wait()
                pltpu.make_async_copy(att2[b], den_acc.at[od], sa[b]).wait()

            pltpu.make_async_copy(sp_hbm.at[sidx_vec], srows[b], gs[b]).wait()
            pltpu.make_async_copy(dp_hbm.at[didx_vec], drows[b], gd[b]).wait()

            compute_group(b)

            pltpu.async_copy(msg[b], num_acc.at[didx_vec], sm[b], add=True)
            pltpu.async_copy(att2[b], den_acc.at[didx_vec], sa[b], add=True)

            @pl.when(gb + NB < NGP)
            def _():
                ns = sidx_buf[gb + NB]
                nd = didx_buf[gb + NB]
                pltpu.async_copy(sp_hbm.at[ns], srows[b], gs[b])
                pltpu.async_copy(dp_hbm.at[nd], drows[b], gd[b])

    for b in range(NB):
        od = didx_buf[NGP - NB + b]
        pltpu.make_async_copy(msg[b], num_acc.at[od], sm[b]).wait()
        pltpu.make_async_copy(att2[b], den_acc.at[od], sa[b]).wait()

    plsc.subcore_barrier()

    @pl.loop(0, NCHUNK)
    def _(c):
        @pl.when(c % NS == sid)
        def _():
            pltpu.sync_copy(num_acc.at[pl.ds(c * ZCH, ZCH)], zbuf)
            pltpu.sync_copy(zbuf, num_out.at[cid, pl.ds(c * ZCH, ZCH)])
            pltpu.sync_copy(den_acc.at[pl.ds(c * ZCH, ZCH)], dzbuf)
            pltpu.sync_copy(dzbuf, den_out.at[cid, pl.ds(c * ZCH, ZCH)])


_HEAD_BCAST = np.zeros((16, 128), np.float32)
for _h in range(H):
    _HEAD_BCAST[_h, _h * D:(_h + 1) * D] = 1.0


@jax.jit
def kernel(x, edge_index, W_src, W_dst, double_attn, bias, prelu_a):
    src2d = jnp.concatenate(
        [edge_index[0].reshape(NW, NGROUPS, 16),
         jnp.zeros((NW, 1, 16), jnp.int32)], axis=1)
    dst2d = jnp.concatenate(
        [edge_index[1].reshape(NW, NGROUPS, 16),
         jnp.full((NW, 1, 16), N, jnp.int32)], axis=1)
    a_flat = double_attn.reshape(H * D)

    PB = 400
    sp, dp = pl.pallas_call(
        _proj_body,
        grid=(N // PB,),
        in_specs=[
            pl.BlockSpec((PB, F), lambda i: (i, 0)),
            pl.BlockSpec((H * D, F), lambda i: (0, 0)),
            pl.BlockSpec((H * D, F), lambda i: (0, 0)),
        ],
        out_specs=[
            pl.BlockSpec((PB, H * D), lambda i: (i, 0)),
            pl.BlockSpec((PB, H * D), lambda i: (i, 0)),
        ],
        out_shape=[
            jax.ShapeDtypeStruct((N, H * D), jnp.float32),
            jax.ShapeDtypeStruct((N, H * D), jnp.float32),
        ],
    )(x, W_src, W_dst)

    mesh = plsc.VectorSubcoreMesh(core_axis_name="c", subcore_axis_name="s")
    cp = pltpu.CompilerParams()
    if "needs_layout_passes" in pltpu.CompilerParams.__dataclass_fields__:
        cp = dataclasses.replace(cp, needs_layout_passes=False)
    if "use_tc_tiling_on_sc" in pltpu.CompilerParams.__dataclass_fields__:
        cp = dataclasses.replace(cp, use_tc_tiling_on_sc=False)
    sc_kernel = functools.partial(
        pl.kernel,
        compiler_params=cp,
        out_type=[
            jax.ShapeDtypeStruct((NC, NP, H * D), jnp.float32),
            jax.ShapeDtypeStruct((NC, NP, 16), jnp.float32),
        ],
        mesh=mesh,
        scratch_types=[
            pltpu.VMEM((NGP, 16), jnp.int32),
            pltpu.VMEM((NGP, 16), jnp.int32),
            pltpu.VMEM((G, H * D), jnp.float32),
            pltpu.VMEM((G, H * D), jnp.float32),
            pltpu.VMEM((G, H * D), jnp.float32),
            pltpu.VMEM((G, H * D), jnp.float32),
            pltpu.VMEM((G, H * D), jnp.float32),
            pltpu.VMEM((G, H * D), jnp.float32),
            pltpu.VMEM((G, 16), jnp.float32),
            pltpu.VMEM((G, 16), jnp.float32),
            pltpu.VMEM((H * 256,), jnp.float32),
            pltpu.VMEM((H * D,), jnp.float32),
            pltpu.VMEM((ZCH, H * D), jnp.float32),
            pltpu.VMEM((ZCH, 16), jnp.float32),
            pltpu.VMEM_SHARED((NP, H * D), jnp.float32),
            pltpu.VMEM_SHARED((NP, 16), jnp.float32),
            pltpu.SemaphoreType.DMA,
            pltpu.SemaphoreType.DMA,
            pltpu.SemaphoreType.DMA,
            pltpu.SemaphoreType.DMA,
            pltpu.SemaphoreType.DMA,
            pltpu.SemaphoreType.DMA,
            pltpu.SemaphoreType.DMA,
            pltpu.SemaphoreType.DMA,
        ],
    )(_gat_edge_kernel)
    num_part, den_part = sc_kernel(sp, dp, src2d, dst2d, a_flat)
    num_part = num_part[:, :N]
    den_part = den_part[:, :N]

    bias_row = bias.reshape(1, H * D)
    pa_row = jnp.broadcast_to(prelu_a.reshape(1, 1), (1, H * D))
    m = jnp.asarray(_HEAD_BCAST)
    out = pl.pallas_call(
        _final_body,
        grid=(N // PB,),
        in_specs=[
            pl.BlockSpec((NC, PB, H * D), lambda i: (0, i, 0)),
            pl.BlockSpec((NC, PB, 16), lambda i: (0, i, 0)),
            pl.BlockSpec((PB, F), lambda i: (i, 0)),
            pl.BlockSpec((1, H * D), lambda i: (0, 0)),
            pl.BlockSpec((16, H * D), lambda i: (0, 0)),
            pl.BlockSpec((1, H * D), lambda i: (0, 0)),
        ],
        out_specs=pl.BlockSpec((PB, H * D), lambda i: (i, 0)),
        out_shape=jax.ShapeDtypeStruct((N, H * D), jnp.float32),
    )(num_part, den_part, x, bias_row, m, pa_row)
    return out
